# baseline (speedup 1.0000x reference)
import jax
import jax.numpy as jnp
from jax import lax
from jax.experimental import pallas as pl
from jax.experimental.pallas import tpu as pltpu
from jax.experimental.pallas import tpu_sc as plsc

N = 10000
E = 160000
D_IN = 256
D_HID = 256
NCLS = 40
NCP = 48
NCP2 = 64

NC, NS = 2, 16
G = 128
EPAD = NC * NS * 40 * G
NGRP = EPAD // G
FEAT_GPT = NGRP // NS
DEG_GPT = NGRP // (NC * NS)
ACC_ROWS = 10240
ZCHUNK = ACC_ROWS // NS
OCHUNK = 624
OLAST = N - (NS - 1) * OCHUNK
CH = 8
CH2 = 5

def _sc1_body(xbf, srcf, dstg, zbf, z16, ones_in,
              m1, dg,
              acc, dacc, sidxs, didx, rows0, rows1, ones,
              gsem0, gsem1, ssem0, ssem1, dsem):
    c = lax.axis_index("c")
    s = lax.axis_index("s")

    pltpu.sync_copy(zbf.at[pl.ds(s * ZCHUNK, ZCHUNK)],
                    acc.at[pl.ds(s * ZCHUNK, ZCHUNK)])
    pltpu.sync_copy(z16.at[pl.ds(s * ZCHUNK, ZCHUNK)],
                    dacc.at[pl.ds(s * ZCHUNK, ZCHUNK)])

    pltpu.sync_copy(ones_in, ones)
    plsc.subcore_barrier()

    wid = c * NS + s
    baseg = wid * DEG_GPT
    basee = baseg * G
    rbufs = (rows0, rows1)
    gsems = (gsem0, gsem1)
    ssems = (ssem0, ssem1)

    def _degslab(blk, carry):
        pltpu.sync_copy(dstg.at[pl.ds(baseg + blk * CH, CH)], didx)
        descs = [pltpu.async_copy(ones, dacc.at[didx.at[j]], dsem, add=True)
                 for j in range(CH)]
        for d in descs:
            d.wait()
        return carry

    lax.fori_loop(0, DEG_GPT // CH, _degslab, 0)

    def _slab(blk, carry):
        pltpu.sync_copy(dstg.at[pl.ds(baseg + blk * CH, CH)], didx)
        pltpu.sync_copy(srcf.at[pl.ds(basee + blk * CH * G, CH * G)], sidxs)
        gd = [None, None]
        sd = [None, None]
        gd[0] = pltpu.async_copy(xbf.at[sidxs.at[pl.ds(0, G)]],
                                 rbufs[0], gsems[0])
        for j in range(CH):
            b = j & 1
            nb = (j + 1) & 1
            if j + 1 < CH:
                if sd[nb] is not None:
                    sd[nb].wait()
                gd[nb] = pltpu.async_copy(
                    xbf.at[sidxs.at[pl.ds((j + 1) * G, G)]],
                    rbufs[nb], gsems[nb])
            gd[b].wait()
            sd[b] = pltpu.async_copy(rbufs[b], acc.at[didx.at[j]],
                                     ssems[b], add=True)
        sd[0].wait()
        sd[1].wait()
        return carry

    lax.fori_loop(0, DEG_GPT // CH, _slab, 0)
    plsc.subcore_barrier()

    @pl.when(s < NS - 1)
    def _():
        pltpu.sync_copy(acc.at[pl.ds(s * OCHUNK, OCHUNK)],
                        m1.at[pl.ds(c * N + s * OCHUNK, OCHUNK)])
        pltpu.sync_copy(dacc.at[pl.ds(s * OCHUNK, OCHUNK)],
                        dg.at[pl.ds(c * N + s * OCHUNK, OCHUNK)])

    @pl.when(s == NS - 1)
    def _():
        pltpu.sync_copy(acc.at[pl.ds((NS - 1) * OCHUNK, OLAST)],
                        m1.at[pl.ds(c * N + (NS - 1) * OCHUNK, OLAST)])
        pltpu.sync_copy(dacc.at[pl.ds((NS - 1) * OCHUNK, OLAST)],
                        dg.at[pl.ds(c * N + (NS - 1) * OCHUNK, OLAST)])


_sc_cache = {}


def _sc1(*args):
    k = _sc_cache.get("sc1")
    if k is None:
        mesh = plsc.VectorSubcoreMesh(core_axis_name="c", subcore_axis_name="s")
        k = _sc_cache["sc1"] = pl.kernel(
            _sc1_body,
            out_type=[jax.ShapeDtypeStruct((2 * N, D_IN), jnp.bfloat16),
                      jax.ShapeDtypeStruct((2 * N, 16), jnp.float32)],
            mesh=mesh,
            scratch_types=[
                pltpu.VMEM_SHARED((ACC_ROWS, D_IN), jnp.bfloat16),
                pltpu.VMEM_SHARED((ACC_ROWS, 16), jnp.float32),
                pltpu.VMEM((CH * G,), jnp.int32),
                pltpu.VMEM((CH, G), jnp.int32),
                pltpu.VMEM((G, D_IN), jnp.bfloat16),
                pltpu.VMEM((G, D_IN), jnp.bfloat16),
                pltpu.VMEM((G, 16), jnp.float32),
                pltpu.SemaphoreType.DMA,
                pltpu.SemaphoreType.DMA,
                pltpu.SemaphoreType.DMA,
                pltpu.SemaphoreType.DMA,
                pltpu.SemaphoreType.DMA,
            ],
            compiler_params=pltpu.CompilerParams(use_tc_tiling_on_sc=False),
        )
    return k(*args)


def _sc2_body(p2, srcf, dstg, z48,
              m2,
              acc, sidx0, sidx1, didx, rows0, rows1,
              gsem0, gsem1, ssem0, ssem1):
    c = lax.axis_index("c")
    s = lax.axis_index("s")

    pltpu.sync_copy(z48.at[pl.ds(s * ZCHUNK, ZCHUNK)],
                    acc.at[pl.ds(s * ZCHUNK, ZCHUNK)])
    plsc.subcore_barrier()

    wid = c * NS + s
    base = wid * DEG_GPT
    basee = base * G
    SS = CH2 * G
    rbufs = (rows0, rows1)
    sibufs = (sidx0, sidx1)
    gsems = (gsem0, gsem1)
    ssems = (ssem0, ssem1)
    NCHUNK = DEG_GPT // CH2
    gd = [None, None]

    def _gather(ch, b):
        pltpu.sync_copy(srcf.at[pl.ds(basee + ch * SS, SS)], sibufs[b])
        gd[b] = pltpu.async_copy(p2.at[sibufs[b]], rbufs[b], gsems[b])

    _gather(0, 0)
    for ch in range(NCHUNK):
        b = ch & 1
        if ch + 1 < NCHUNK:
            _gather(ch + 1, (ch + 1) & 1)
        gd[b].wait()
        pltpu.sync_copy(dstg.at[pl.ds(base + ch * CH2, CH2)], didx)
        sds = [pltpu.async_copy(rbufs[b].at[pl.ds(j * G, G)],
                                acc.at[didx.at[j]], ssems[j & 1], add=True)
               for j in range(CH2)]
        for d in sds:
            d.wait()
    plsc.subcore_barrier()

    @pl.when(s < NS - 1)
    def _():
        pltpu.sync_copy(acc.at[pl.ds(s * OCHUNK, OCHUNK)],
                        m2.at[pl.ds(c * N + s * OCHUNK, OCHUNK)])

    @pl.when(s == NS - 1)
    def _():
        pltpu.sync_copy(acc.at[pl.ds((NS - 1) * OCHUNK, OLAST)],
                        m2.at[pl.ds(c * N + (NS - 1) * OCHUNK, OLAST)])


def _sc2(*args):
    k = _sc_cache.get("sc2")
    if k is None:
        mesh = plsc.VectorSubcoreMesh(core_axis_name="c", subcore_axis_name="s")
        k = _sc_cache["sc2"] = pl.kernel(
            _sc2_body,
            out_type=jax.ShapeDtypeStruct((2 * N, NCP2), jnp.bfloat16),
            mesh=mesh,
            scratch_types=[
                pltpu.VMEM_SHARED((ACC_ROWS, NCP2), jnp.bfloat16),
                pltpu.VMEM((CH2 * G,), jnp.int32),
                pltpu.VMEM((CH2 * G,), jnp.int32),
                pltpu.VMEM((CH2, G), jnp.int32),
                pltpu.VMEM((CH2 * G, NCP2), jnp.bfloat16),
                pltpu.VMEM((CH2 * G, NCP2), jnp.bfloat16),
                pltpu.SemaphoreType.DMA,
                pltpu.SemaphoreType.DMA,
                pltpu.SemaphoreType.DMA,
                pltpu.SemaphoreType.DMA,
            ],
            compiler_params=pltpu.CompilerParams(use_tc_tiling_on_sc=False),
        )
    return k(*args)


_MB = 1000


def _tc2_body(x, m1, dg, w1s, w1n, b1r, w2, b2r, s2, p2):
    dgb = dg[:]
    deg = dgb[0] + dgb[1]
    rdeg = 1.0 / jnp.maximum(deg[:, 0:1], 1.0)
    m1b = m1[:].astype(jnp.float32)
    meanx = (m1b[0] + m1b[1]) * rdeg
    h = jnp.dot(x[:], w1s[:], preferred_element_type=jnp.float32)
    h = h + jnp.dot(meanx, w1n[:], preferred_element_type=jnp.float32)
    h = jnp.maximum(h + b1r[:], 0.0)
    z = jnp.dot(h, w2[:], preferred_element_type=jnp.float32) + b2r[:]
    s2[:] = z[:, :NCP]
    p2[:] = z[:, NCP:].astype(jnp.bfloat16)


def _tc3_body(s2, m2, dg, out):
    dgb = dg[:]
    deg = dgb[0] + dgb[1]
    rdeg = 1.0 / jnp.maximum(deg[:, 0:1], 1.0)
    m2b = m2[:].astype(jnp.float32)
    m2s = (m2b[0] + m2b[1])[:, :NCP]
    out[:] = s2[:] + m2s * rdeg


def kernel(input_matrix, adj, W1_self, W1_neigh, b1, W2_self, W2_neigh, b2):
    f32 = jnp.float32
    x = input_matrix.astype(f32)

    src = adj[0].astype(jnp.int32)
    dst = adj[1].astype(jnp.int32)
    pad = EPAD - E
    srcg = jnp.concatenate([src, jnp.zeros((pad,), jnp.int32)]).reshape(NGRP, G)
    dstg = jnp.concatenate([dst, jnp.full((pad,), N, jnp.int32)]).reshape(NGRP, G)

    bc1 = b1.reshape(1, D_HID)
    zpad = jnp.zeros((D_HID, NCP - NCLS), f32)
    zpad2 = jnp.zeros((D_HID, NCP2 - NCLS), f32)
    w2 = jnp.concatenate([W2_self, zpad, W2_neigh, zpad2], axis=1)
    bc2 = jnp.concatenate(
        [b2, jnp.zeros((NCP + NCP2 - NCLS,), f32)]).reshape(1, NCP + NCP2)

    zbf = jnp.zeros((ACC_ROWS, D_IN), jnp.bfloat16)
    z16 = jnp.zeros((ACC_ROWS, 16), f32)
    z64 = jnp.zeros((ACC_ROWS, NCP2), jnp.bfloat16)

    xbf = x.astype(jnp.bfloat16)
    ones_in = jnp.ones((G, 16), f32)
    m1, dg = _sc1(xbf, srcg.reshape(-1), dstg, zbf, z16, ones_in)
    m1 = m1.reshape(2, N, D_IN)
    dg = dg.reshape(2, N, 16)

    s2, p2 = pl.pallas_call(
        _tc2_body,
        grid=(N // _MB,),
        in_specs=[pl.BlockSpec((_MB, D_IN), lambda i: (i, 0)),
                  pl.BlockSpec((2, _MB, D_IN), lambda i: (0, i, 0)),
                  pl.BlockSpec((2, _MB, 16), lambda i: (0, i, 0)),
                  pl.BlockSpec((D_IN, D_HID), lambda i: (0, 0)),
                  pl.BlockSpec((D_IN, D_HID), lambda i: (0, 0)),
                  pl.BlockSpec((1, D_HID), lambda i: (0, 0)),
                  pl.BlockSpec((D_HID, NCP + NCP2), lambda i: (0, 0)),
                  pl.BlockSpec((1, NCP + NCP2), lambda i: (0, 0))],
        out_specs=[pl.BlockSpec((_MB, NCP), lambda i: (i, 0)),
                   pl.BlockSpec((_MB, NCP2), lambda i: (i, 0))],
        out_shape=[jax.ShapeDtypeStruct((N, NCP), f32),
                   jax.ShapeDtypeStruct((N, NCP2), jnp.bfloat16)],
    )(x, m1, dg, W1_self, W1_neigh, bc1, w2, bc2)

    m2 = _sc2(p2, srcg.reshape(-1), dstg, z64).reshape(2, N, NCP2)

    out = pl.pallas_call(
        _tc3_body,
        grid=(N // _MB,),
        in_specs=[pl.BlockSpec((_MB, NCP), lambda i: (i, 0)),
                  pl.BlockSpec((2, _MB, NCP2), lambda i: (0, i, 0)),
                  pl.BlockSpec((2, _MB, 16), lambda i: (0, i, 0))],
        out_specs=pl.BlockSpec((_MB, NCP), lambda i: (i, 0)),
        out_shape=jax.ShapeDtypeStruct((N, NCP), f32),
    )(s2, m2, dg)

    return out[:, :NCLS]


# speedup vs baseline: 1.0324x; 1.0324x over previous
import jax
import jax.numpy as jnp
from jax import lax
from jax.experimental import pallas as pl
from jax.experimental.pallas import tpu as pltpu
from jax.experimental.pallas import tpu_sc as plsc

N = 10000
E = 160000
D_IN = 256
D_HID = 256
NCLS = 40
NCP = 48

NC, NS = 2, 16
G = 128
EPAD = NC * NS * 40 * G
NGRP = EPAD // G
FEAT_GPT = NGRP // NS
DEG_GPT = NGRP // (NC * NS)
ACC_ROWS = 10240
ZCHUNK = ACC_ROWS // NS
OCHUNK = 624
OLAST = N - (NS - 1) * OCHUNK
CH = 8
CH2 = 5

def _sc1_body(xbf, srcf, dstg, zbf, z16, ones_in,
              m1, dg,
              acc, dacc, sidxs, didx, rows0, rows1, ones,
              gsem0, gsem1, ssem0, ssem1, dsem):
    c = lax.axis_index("c")
    s = lax.axis_index("s")

    pltpu.sync_copy(zbf.at[pl.ds(s * ZCHUNK, ZCHUNK)],
                    acc.at[pl.ds(s * ZCHUNK, ZCHUNK)])
    pltpu.sync_copy(z16.at[pl.ds(s * ZCHUNK, ZCHUNK)],
                    dacc.at[pl.ds(s * ZCHUNK, ZCHUNK)])

    pltpu.sync_copy(ones_in, ones)
    plsc.subcore_barrier()

    wid = c * NS + s
    baseg = wid * DEG_GPT
    basee = baseg * G
    rbufs = (rows0, rows1)
    gsems = (gsem0, gsem1)
    ssems = (ssem0, ssem1)

    def _degslab(blk, carry):
        pltpu.sync_copy(dstg.at[pl.ds(baseg + blk * CH, CH)], didx)
        descs = [pltpu.async_copy(ones, dacc.at[didx.at[j]], dsem, add=True)
                 for j in range(CH)]
        for d in descs:
            d.wait()
        return carry

    lax.fori_loop(0, DEG_GPT // CH, _degslab, 0)

    def _slab(blk, carry):
        pltpu.sync_copy(dstg.at[pl.ds(baseg + blk * CH, CH)], didx)
        pltpu.sync_copy(srcf.at[pl.ds(basee + blk * CH * G, CH * G)], sidxs)
        gd = [None, None]
        sd = [None, None]
        gd[0] = pltpu.async_copy(xbf.at[sidxs.at[pl.ds(0, G)]],
                                 rbufs[0], gsems[0])
        for j in range(CH):
            b = j & 1
            nb = (j + 1) & 1
            if j + 1 < CH:
                if sd[nb] is not None:
                    sd[nb].wait()
                gd[nb] = pltpu.async_copy(
                    xbf.at[sidxs.at[pl.ds((j + 1) * G, G)]],
                    rbufs[nb], gsems[nb])
            gd[b].wait()
            sd[b] = pltpu.async_copy(rbufs[b], acc.at[didx.at[j]],
                                     ssems[b], add=True)
        sd[0].wait()
        sd[1].wait()
        return carry

    lax.fori_loop(0, DEG_GPT // CH, _slab, 0)
    plsc.subcore_barrier()

    @pl.when(s < NS - 1)
    def _():
        pltpu.sync_copy(acc.at[pl.ds(s * OCHUNK, OCHUNK)],
                        m1.at[pl.ds(c * N + s * OCHUNK, OCHUNK)])
        pltpu.sync_copy(dacc.at[pl.ds(s * OCHUNK, OCHUNK)],
                        dg.at[pl.ds(c * N + s * OCHUNK, OCHUNK)])

    @pl.when(s == NS - 1)
    def _():
        pltpu.sync_copy(acc.at[pl.ds((NS - 1) * OCHUNK, OLAST)],
                        m1.at[pl.ds(c * N + (NS - 1) * OCHUNK, OLAST)])
        pltpu.sync_copy(dacc.at[pl.ds((NS - 1) * OCHUNK, OLAST)],
                        dg.at[pl.ds(c * N + (NS - 1) * OCHUNK, OLAST)])


_sc_cache = {}


def _sc1(*args):
    k = _sc_cache.get("sc1")
    if k is None:
        mesh = plsc.VectorSubcoreMesh(core_axis_name="c", subcore_axis_name="s")
        k = _sc_cache["sc1"] = pl.kernel(
            _sc1_body,
            out_type=[jax.ShapeDtypeStruct((2 * N, D_IN), jnp.bfloat16),
                      jax.ShapeDtypeStruct((2 * N, 16), jnp.float32)],
            mesh=mesh,
            scratch_types=[
                pltpu.VMEM_SHARED((ACC_ROWS, D_IN), jnp.bfloat16),
                pltpu.VMEM_SHARED((ACC_ROWS, 16), jnp.float32),
                pltpu.VMEM((CH * G,), jnp.int32),
                pltpu.VMEM((CH, G), jnp.int32),
                pltpu.VMEM((G, D_IN), jnp.bfloat16),
                pltpu.VMEM((G, D_IN), jnp.bfloat16),
                pltpu.VMEM((G, 16), jnp.float32),
                pltpu.SemaphoreType.DMA,
                pltpu.SemaphoreType.DMA,
                pltpu.SemaphoreType.DMA,
                pltpu.SemaphoreType.DMA,
                pltpu.SemaphoreType.DMA,
            ],
            compiler_params=pltpu.CompilerParams(use_tc_tiling_on_sc=False),
        )
    return k(*args)


def _sc2_body(p2, srcf, dstg, z48,
              m2,
              acc, sidx0, sidx1, didx, rows0, rows1,
              gsem0, gsem1, ssem0, ssem1):
    c = lax.axis_index("c")
    s = lax.axis_index("s")

    pltpu.sync_copy(z48.at[pl.ds(s * ZCHUNK, ZCHUNK)],
                    acc.at[pl.ds(s * ZCHUNK, ZCHUNK)])
    plsc.subcore_barrier()

    wid = c * NS + s
    base = wid * DEG_GPT
    basee = base * G
    SS = CH2 * G
    rbufs = (rows0, rows1)
    sibufs = (sidx0, sidx1)
    gsems = (gsem0, gsem1)
    ssems = (ssem0, ssem1)
    NCHUNK = DEG_GPT // CH2
    gd = [None, None]

    def _gather(ch, b):
        pltpu.sync_copy(srcf.at[pl.ds(basee + ch * SS, SS)], sibufs[b])
        gd[b] = pltpu.async_copy(p2.at[sibufs[b]], rbufs[b], gsems[b])

    _gather(0, 0)
    for ch in range(NCHUNK):
        b = ch & 1
        if ch + 1 < NCHUNK:
            _gather(ch + 1, (ch + 1) & 1)
        gd[b].wait()
        pltpu.sync_copy(dstg.at[pl.ds(base + ch * CH2, CH2)], didx)
        sds = [pltpu.async_copy(rbufs[b].at[pl.ds(j * G, G)],
                                acc.at[didx.at[j]], ssems[j & 1], add=True)
               for j in range(CH2)]
        for d in sds:
            d.wait()
    plsc.subcore_barrier()

    @pl.when(s < NS - 1)
    def _():
        pltpu.sync_copy(acc.at[pl.ds(s * OCHUNK, OCHUNK)],
                        m2.at[pl.ds(c * N + s * OCHUNK, OCHUNK)])

    @pl.when(s == NS - 1)
    def _():
        pltpu.sync_copy(acc.at[pl.ds((NS - 1) * OCHUNK, OLAST)],
                        m2.at[pl.ds(c * N + (NS - 1) * OCHUNK, OLAST)])


def _sc2(*args):
    k = _sc_cache.get("sc2")
    if k is None:
        mesh = plsc.VectorSubcoreMesh(core_axis_name="c", subcore_axis_name="s")
        k = _sc_cache["sc2"] = pl.kernel(
            _sc2_body,
            out_type=jax.ShapeDtypeStruct((2 * N, NCP), jnp.float32),
            mesh=mesh,
            scratch_types=[
                pltpu.VMEM_SHARED((ACC_ROWS, NCP), jnp.float32),
                pltpu.VMEM((CH2 * G,), jnp.int32),
                pltpu.VMEM((CH2 * G,), jnp.int32),
                pltpu.VMEM((CH2, G), jnp.int32),
                pltpu.VMEM((CH2 * G, NCP), jnp.float32),
                pltpu.VMEM((CH2 * G, NCP), jnp.float32),
                pltpu.SemaphoreType.DMA,
                pltpu.SemaphoreType.DMA,
                pltpu.SemaphoreType.DMA,
                pltpu.SemaphoreType.DMA,
            ],
            compiler_params=pltpu.CompilerParams(use_tc_tiling_on_sc=False),
        )
    return k(*args)


_MB = 1000


def _tc2_body(x, m1, dg, w1s, w1n, b1r, w2, b2r, s2, p2):
    dgb = dg[:]
    deg = dgb[0] + dgb[1]
    rdeg = 1.0 / jnp.maximum(deg[:, 0:1], 1.0)
    m1b = m1[:].astype(jnp.float32)
    meanx = (m1b[0] + m1b[1]) * rdeg
    h = jnp.dot(x[:], w1s[:], preferred_element_type=jnp.float32)
    h = h + jnp.dot(meanx, w1n[:], preferred_element_type=jnp.float32)
    h = jnp.maximum(h + b1r[:], 0.0)
    z = jnp.dot(h, w2[:], preferred_element_type=jnp.float32) + b2r[:]
    s2[:] = z[:, :NCP]
    p2[:] = z[:, NCP:]


def _tc3_body(s2, m2, dg, out):
    dgb = dg[:]
    deg = dgb[0] + dgb[1]
    rdeg = 1.0 / jnp.maximum(deg[:, 0:1], 1.0)
    m2b = m2[:]
    out[:] = s2[:] + (m2b[0] + m2b[1]) * rdeg


def kernel(input_matrix, adj, W1_self, W1_neigh, b1, W2_self, W2_neigh, b2):
    f32 = jnp.float32
    x = input_matrix.astype(f32)

    src = adj[0].astype(jnp.int32)
    dst = adj[1].astype(jnp.int32)
    pad = EPAD - E
    srcg = jnp.concatenate([src, jnp.zeros((pad,), jnp.int32)]).reshape(NGRP, G)
    dstg = jnp.concatenate([dst, jnp.full((pad,), N, jnp.int32)]).reshape(NGRP, G)

    bc1 = b1.reshape(1, D_HID)
    zpad = jnp.zeros((D_HID, NCP - NCLS), f32)
    w2 = jnp.concatenate([W2_self, zpad, W2_neigh, zpad], axis=1)
    bc2 = jnp.concatenate(
        [b2, jnp.zeros((2 * NCP - NCLS,), f32)]).reshape(1, 2 * NCP)

    zbf = jnp.zeros((ACC_ROWS, D_IN), jnp.bfloat16)
    z16 = jnp.zeros((ACC_ROWS, 16), f32)
    z48 = jnp.zeros((ACC_ROWS, NCP), f32)

    xbf = x.astype(jnp.bfloat16)
    ones_in = jnp.ones((G, 16), f32)
    m1, dg = _sc1(xbf, srcg.reshape(-1), dstg, zbf, z16, ones_in)
    m1 = m1.reshape(2, N, D_IN)
    dg = dg.reshape(2, N, 16)

    s2, p2 = pl.pallas_call(
        _tc2_body,
        grid=(N // _MB,),
        in_specs=[pl.BlockSpec((_MB, D_IN), lambda i: (i, 0)),
                  pl.BlockSpec((2, _MB, D_IN), lambda i: (0, i, 0)),
                  pl.BlockSpec((2, _MB, 16), lambda i: (0, i, 0)),
                  pl.BlockSpec((D_IN, D_HID), lambda i: (0, 0)),
                  pl.BlockSpec((D_IN, D_HID), lambda i: (0, 0)),
                  pl.BlockSpec((1, D_HID), lambda i: (0, 0)),
                  pl.BlockSpec((D_HID, 2 * NCP), lambda i: (0, 0)),
                  pl.BlockSpec((1, 2 * NCP), lambda i: (0, 0))],
        out_specs=[pl.BlockSpec((_MB, NCP), lambda i: (i, 0)),
                   pl.BlockSpec((_MB, NCP), lambda i: (i, 0))],
        out_shape=[jax.ShapeDtypeStruct((N, NCP), f32),
                   jax.ShapeDtypeStruct((N, NCP), f32)],
    )(x, m1, dg, W1_self, W1_neigh, bc1, w2, bc2)

    m2 = _sc2(p2, srcg.reshape(-1), dstg, z48).reshape(2, N, NCP)

    out = pl.pallas_call(
        _tc3_body,
        grid=(N // _MB,),
        in_specs=[pl.BlockSpec((_MB, NCP), lambda i: (i, 0)),
                  pl.BlockSpec((2, _MB, NCP), lambda i: (0, i, 0)),
                  pl.BlockSpec((2, _MB, 16), lambda i: (0, i, 0))],
        out_specs=pl.BlockSpec((_MB, NCP), lambda i: (i, 0)),
        out_shape=jax.ShapeDtypeStruct((N, NCP), f32),
    )(s2, m2, dg)

    return out[:, :NCLS]
